# baseline (speedup 1.0000x reference)
import functools

import jax
import jax.numpy as jnp
from jax.experimental import pallas as pl

D_MODEL = 2048
HIDDEN = 128
N_EXPERTS = 16
TOP_K = 2

TOKEN_TILE = 512


def _router_body(x_ref, w1_ref, w2_ref, logits_ref, w_ref, idx_ref):
    x = x_ref[...]
    h = jax.lax.dot_general(
        x, w1_ref[...], (((1,), (0,)), ((), ())),
        preferred_element_type=jnp.float32,
    )
    h = h * (1.0 / (1.0 + jnp.exp(-h)))
    logits = jax.lax.dot_general(
        h, w2_ref[...], (((1,), (0,)), ((), ())),
        preferred_element_type=jnp.float32,
    )
    logits_ref[...] = logits

    t = logits.shape[0]
    iota = jax.lax.broadcasted_iota(jnp.int32, (t, N_EXPERTS), 1)
    m1 = jnp.max(logits, axis=1, keepdims=True)
    i1 = jnp.min(jnp.where(logits == m1, iota, N_EXPERTS), axis=1, keepdims=True)
    masked = jnp.where(iota == i1, -jnp.inf, logits)
    m2 = jnp.max(masked, axis=1, keepdims=True)
    i2 = jnp.min(jnp.where(masked == m2, iota, N_EXPERTS), axis=1, keepdims=True)

    e = jnp.exp(m2 - m1)
    denom = 1.0 + e
    w_ref[...] = jnp.concatenate([1.0 / denom, e / denom], axis=1)
    idx_ref[...] = jnp.concatenate([i1, i2], axis=1)


@functools.partial(jax.jit, static_argnames=("interpret",))
def _router(x, w1, w2, interpret=False):
    n_tok = x.shape[0]
    grid = (n_tok // TOKEN_TILE,)
    return pl.pallas_call(
        _router_body,
        grid=grid,
        in_specs=[
            pl.BlockSpec((TOKEN_TILE, D_MODEL), lambda i: (i, 0)),
            pl.BlockSpec((D_MODEL, HIDDEN), lambda i: (0, 0)),
            pl.BlockSpec((HIDDEN, N_EXPERTS), lambda i: (0, 0)),
        ],
        out_specs=[
            pl.BlockSpec((TOKEN_TILE, N_EXPERTS), lambda i: (i, 0)),
            pl.BlockSpec((TOKEN_TILE, TOP_K), lambda i: (i, 0)),
            pl.BlockSpec((TOKEN_TILE, TOP_K), lambda i: (i, 0)),
        ],
        out_shape=[
            jax.ShapeDtypeStruct((n_tok, N_EXPERTS), jnp.float32),
            jax.ShapeDtypeStruct((n_tok, TOP_K), jnp.float32),
            jax.ShapeDtypeStruct((n_tok, TOP_K), jnp.int32),
        ],
        interpret=interpret,
    )(x, w1, w2)


def kernel(hidden_states, W1, W2):
    b, s, d = hidden_states.shape
    x = hidden_states.reshape(b * s, d)
    logits, w, idx = _router(x, W1, W2)
    return (
        w.reshape(b, s, TOP_K),
        idx.reshape(b, s, TOP_K),
        logits.reshape(b, s, N_EXPERTS),
    )


# speedup vs baseline: 1.5552x; 1.5552x over previous
import functools

import jax
import jax.numpy as jnp
from jax.experimental import pallas as pl

D_MODEL = 2048
HIDDEN = 128
N_EXPERTS = 16
TOP_K = 2

TOKEN_TILE = 1024


def _router_body(x_ref, w1_ref, w2_ref, logits_ref, w_ref, idx_ref):
    x = x_ref[...]
    h = jax.lax.dot_general(
        x, w1_ref[...], (((1,), (0,)), ((), ())),
        preferred_element_type=jnp.float32,
    )
    h = h * (1.0 / (1.0 + jnp.exp(-h)))
    logits = jax.lax.dot_general(
        h, w2_ref[...], (((1,), (0,)), ((), ())),
        preferred_element_type=jnp.float32,
    )
    logits_ref[...] = logits
    logits_t = jax.lax.dot_general(
        w2_ref[...], h, (((0,), (1,)), ((), ())),
        preferred_element_type=jnp.float32,
    )

    t = logits_t.shape[1]
    iota = jax.lax.broadcasted_iota(jnp.int32, (N_EXPERTS, t), 0)
    m1 = jnp.max(logits_t, axis=0, keepdims=True)
    i1 = jnp.min(jnp.where(logits_t == m1, iota, N_EXPERTS), axis=0, keepdims=True)
    masked = jnp.where(iota == i1, -jnp.inf, logits_t)
    m2 = jnp.max(masked, axis=0, keepdims=True)
    i2 = jnp.min(jnp.where(masked == m2, iota, N_EXPERTS), axis=0, keepdims=True)

    e = jnp.exp(m2 - m1)
    denom = 1.0 + e
    w_ref[...] = jnp.concatenate([1.0 / denom, e / denom], axis=0)
    idx_ref[...] = jnp.concatenate([i1, i2], axis=0)


@functools.partial(jax.jit, static_argnames=("interpret",))
def _router(x, w1, w2, interpret=False):
    n_tok = x.shape[0]
    grid = (n_tok // TOKEN_TILE,)
    return pl.pallas_call(
        _router_body,
        grid=grid,
        in_specs=[
            pl.BlockSpec((TOKEN_TILE, D_MODEL), lambda i: (i, 0)),
            pl.BlockSpec((D_MODEL, HIDDEN), lambda i: (0, 0)),
            pl.BlockSpec((HIDDEN, N_EXPERTS), lambda i: (0, 0)),
        ],
        out_specs=[
            pl.BlockSpec((TOKEN_TILE, N_EXPERTS), lambda i: (i, 0)),
            pl.BlockSpec((TOP_K, TOKEN_TILE), lambda i: (0, i)),
            pl.BlockSpec((TOP_K, TOKEN_TILE), lambda i: (0, i)),
        ],
        out_shape=[
            jax.ShapeDtypeStruct((n_tok, N_EXPERTS), jnp.float32),
            jax.ShapeDtypeStruct((TOP_K, n_tok), jnp.float32),
            jax.ShapeDtypeStruct((TOP_K, n_tok), jnp.int32),
        ],
        interpret=interpret,
    )(x, w1, w2)


def kernel(hidden_states, W1, W2):
    b, s, d = hidden_states.shape
    x = hidden_states.reshape(b * s, d)
    logits, w_t, idx_t = _router(x, W1, W2)
    return (
        w_t.T.reshape(b, s, TOP_K),
        idx_t.T.reshape(b, s, TOP_K),
        logits.reshape(b, s, N_EXPERTS),
    )


# speedup vs baseline: 1.5719x; 1.0107x over previous
import functools

import jax
import jax.numpy as jnp
from jax.experimental import pallas as pl

D_MODEL = 2048
HIDDEN = 128
N_EXPERTS = 16
TOP_K = 2

TOKEN_TILE = 2048


def _router_body(x_ref, w1_ref, w2_ref, logits_ref, w_ref, idx_ref):
    x = x_ref[...]
    h = jax.lax.dot_general(
        x, w1_ref[...], (((1,), (0,)), ((), ())),
        preferred_element_type=jnp.float32,
    )
    h = h * (1.0 / (1.0 + jnp.exp(-h)))
    logits = jax.lax.dot_general(
        h, w2_ref[...], (((1,), (0,)), ((), ())),
        preferred_element_type=jnp.float32,
    )
    logits_ref[...] = logits
    logits_t = jax.lax.dot_general(
        w2_ref[...], h, (((0,), (1,)), ((), ())),
        preferred_element_type=jnp.float32,
    )

    t = logits_t.shape[1]
    iota = jax.lax.broadcasted_iota(jnp.int32, (N_EXPERTS, t), 0)
    m1 = jnp.max(logits_t, axis=0, keepdims=True)
    i1 = jnp.min(jnp.where(logits_t == m1, iota, N_EXPERTS), axis=0, keepdims=True)
    masked = jnp.where(iota == i1, -jnp.inf, logits_t)
    m2 = jnp.max(masked, axis=0, keepdims=True)
    i2 = jnp.min(jnp.where(masked == m2, iota, N_EXPERTS), axis=0, keepdims=True)

    e = jnp.exp(m2 - m1)
    denom = 1.0 + e
    w_ref[...] = jnp.concatenate([1.0 / denom, e / denom], axis=0)
    idx_ref[...] = jnp.concatenate([i1, i2], axis=0)


@functools.partial(jax.jit, static_argnames=("interpret",))
def _router(x, w1, w2, interpret=False):
    n_tok = x.shape[0]
    grid = (n_tok // TOKEN_TILE,)
    return pl.pallas_call(
        _router_body,
        grid=grid,
        in_specs=[
            pl.BlockSpec((TOKEN_TILE, D_MODEL), lambda i: (i, 0)),
            pl.BlockSpec((D_MODEL, HIDDEN), lambda i: (0, 0)),
            pl.BlockSpec((HIDDEN, N_EXPERTS), lambda i: (0, 0)),
        ],
        out_specs=[
            pl.BlockSpec((TOKEN_TILE, N_EXPERTS), lambda i: (i, 0)),
            pl.BlockSpec((TOP_K, TOKEN_TILE), lambda i: (0, i)),
            pl.BlockSpec((TOP_K, TOKEN_TILE), lambda i: (0, i)),
        ],
        out_shape=[
            jax.ShapeDtypeStruct((n_tok, N_EXPERTS), jnp.float32),
            jax.ShapeDtypeStruct((TOP_K, n_tok), jnp.float32),
            jax.ShapeDtypeStruct((TOP_K, n_tok), jnp.int32),
        ],
        interpret=interpret,
    )(x, w1, w2)


def kernel(hidden_states, W1, W2):
    b, s, d = hidden_states.shape
    x = hidden_states.reshape(b * s, d)
    logits, w_t, idx_t = _router(x, W1, W2)
    return (
        w_t.T.reshape(b, s, TOP_K),
        idx_t.T.reshape(b, s, TOP_K),
        logits.reshape(b, s, N_EXPERTS),
    )
